# Initial kernel scaffold; baseline (speedup 1.0000x reference)
import functools

import jax
import jax.numpy as jnp
from jax import lax
from jax.experimental import pallas as pl
from jax.experimental.pallas import tpu as pltpu
from jax.experimental.pallas import tpu_sc as plsc

N_NODES = 10000
N_EDGES = 320000
NC, NS, L = 2, 16, 16
NW = NC * NS
E_PER = N_EDGES // NW
H_PAD = 10240


def _hist_body(a_hbm, b_hbm, out_hbm, a_v, b_v, hist_v):
    wid = lax.axis_index("s") * NC + lax.axis_index("c")
    base = wid * E_PER
    pltpu.sync_copy(a_hbm.at[pl.ds(base, E_PER)], a_v)
    pltpu.sync_copy(b_hbm.at[pl.ds(base, E_PER)], b_v)

    zeros = jnp.zeros((L,), jnp.float32)

    def zero_body(i, c):
        hist_v[pl.ds(i * L, L)] = zeros
        return c

    lax.fori_loop(0, H_PAD // L, zero_body, 0)

    ones = jnp.ones((L,), jnp.float32)

    def scat_body(i, c):
        plsc.addupdate_scatter(hist_v, [a_v[pl.ds(i * L, L)]], ones)
        plsc.addupdate_scatter(hist_v, [b_v[pl.ds(i * L, L)]], ones)
        return c

    lax.fori_loop(0, E_PER // L, scat_body, 0)

    pltpu.sync_copy(hist_v, out_hbm.at[wid])


@jax.jit
def _partial_hist(ref_a, ref_b):
    mesh = plsc.VectorSubcoreMesh(core_axis_name="c", subcore_axis_name="s")
    return pl.kernel(
        _hist_body,
        out_type=jax.ShapeDtypeStruct((NW, H_PAD), jnp.float32),
        mesh=mesh,
        scratch_types=[
            pltpu.VMEM((E_PER,), jnp.int32),
            pltpu.VMEM((E_PER,), jnp.int32),
            pltpu.VMEM((H_PAD,), jnp.float32),
        ],
    )(ref_a, ref_b)


def _norm_body(part_ref, out_ref):
    s = jnp.sum(part_ref[...], axis=0, keepdims=True)
    out_ref[...] = lax.rsqrt(s + 1.0)


@jax.jit
def _norm(part):
    return pl.pallas_call(
        _norm_body,
        out_shape=jax.ShapeDtypeStruct((1, H_PAD), jnp.float32),
    )(part)


def kernel(X, ref_a, ref_b):
    part = _partial_hist(ref_a, ref_b)
    normp = _norm(part)
    return normp[0, :N_NODES].reshape(N_NODES, 1)


# speedup vs baseline: 128.7893x; 128.7893x over previous
import functools

import jax
import jax.numpy as jnp
from jax import lax
from jax.experimental import pallas as pl
from jax.experimental.pallas import tpu as pltpu
from jax.experimental.pallas import tpu_sc as plsc

N_NODES = 10000
N_EDGES = 320000
NC, NS, L = 2, 16, 16
NW = NC * NS
E_PER = N_EDGES // NW
H_PAD = 10240


def _hist_body(a_hbm, b_hbm, out_hbm, a_v, b_v, hist_v):
    wid = lax.axis_index("s") * NC + lax.axis_index("c")
    base = wid * E_PER
    pltpu.sync_copy(a_hbm.at[pl.ds(base, E_PER)], a_v)
    pltpu.sync_copy(b_hbm.at[pl.ds(base, E_PER)], b_v)

    zeros = jnp.zeros((L,), jnp.float32)

    def zero_body(i, c):
        hist_v[pl.ds(i * L, L)] = zeros
        return c

    lax.fori_loop(0, H_PAD // L, zero_body, 0)

    ones = jnp.ones((L,), jnp.float32)

    def scat_body(i, c):
        plsc.addupdate_scatter(hist_v, [a_v[pl.ds(i * L, L)]], ones)
        plsc.addupdate_scatter(hist_v, [b_v[pl.ds(i * L, L)]], ones)
        return c

    lax.fori_loop(0, E_PER // L, scat_body, 0)

    pltpu.sync_copy(hist_v, out_hbm.at[wid])


@jax.jit
def _partial_hist(ref_a, ref_b):
    mesh = plsc.VectorSubcoreMesh(core_axis_name="c", subcore_axis_name="s")
    return pl.kernel(
        _hist_body,
        out_type=jax.ShapeDtypeStruct((NW, H_PAD), jnp.float32),
        mesh=mesh,
        compiler_params=pltpu.CompilerParams(needs_layout_passes=False),
        scratch_types=[
            pltpu.VMEM((E_PER,), jnp.int32),
            pltpu.VMEM((E_PER,), jnp.int32),
            pltpu.VMEM((H_PAD,), jnp.float32),
        ],
    )(ref_a, ref_b)


def _norm_body(part_ref, out_ref):
    s = jnp.sum(part_ref[...], axis=0, keepdims=True)
    out_ref[...] = lax.rsqrt(s + 1.0)


@jax.jit
def _norm(part):
    return pl.pallas_call(
        _norm_body,
        out_shape=jax.ShapeDtypeStruct((1, H_PAD), jnp.float32),
    )(part)


def kernel(X, ref_a, ref_b):
    part = _partial_hist(ref_a, ref_b)
    normp = _norm(part)
    return normp[0, :N_NODES].reshape(N_NODES, 1)


# speedup vs baseline: 137.1106x; 1.0646x over previous
import functools

import jax
import jax.numpy as jnp
from jax import lax
from jax.experimental import pallas as pl
from jax.experimental.pallas import tpu as pltpu
from jax.experimental.pallas import tpu_sc as plsc

N_NODES = 10000
N_EDGES = 320000
NC, NS, L = 2, 16, 16
NW = NC * NS
E_PER = N_EDGES // NW
H_PAD = 10240


def _hist_body(a_hbm, b_hbm, out_hbm, a_v, b_v, hist_v):
    wid = lax.axis_index("s") * NC + lax.axis_index("c")
    base = wid * E_PER
    pltpu.sync_copy(a_hbm.at[pl.ds(base, E_PER)], a_v)
    pltpu.sync_copy(b_hbm.at[pl.ds(base, E_PER)], b_v)

    zeros = jnp.zeros((L,), jnp.float32)

    def zero_body(i, c):
        hist_v[pl.ds(i * L, L)] = zeros
        return c

    lax.fori_loop(0, H_PAD // L, zero_body, 0, unroll=8)

    ones = jnp.ones((L,), jnp.float32)
    UNROLL = 8

    def scat_body(i, c):
        base = i * (L * UNROLL)
        for u in range(UNROLL):
            plsc.addupdate_scatter(hist_v, [a_v[pl.ds(base + u * L, L)]], ones)
            plsc.addupdate_scatter(hist_v, [b_v[pl.ds(base + u * L, L)]], ones)
        return c

    lax.fori_loop(0, E_PER // (L * UNROLL), scat_body, 0)

    pltpu.sync_copy(hist_v, out_hbm.at[wid])


@jax.jit
def _partial_hist(ref_a, ref_b):
    mesh = plsc.VectorSubcoreMesh(core_axis_name="c", subcore_axis_name="s")
    return pl.kernel(
        _hist_body,
        out_type=jax.ShapeDtypeStruct((NW, H_PAD), jnp.float32),
        mesh=mesh,
        compiler_params=pltpu.CompilerParams(needs_layout_passes=False),
        scratch_types=[
            pltpu.VMEM((E_PER,), jnp.int32),
            pltpu.VMEM((E_PER,), jnp.int32),
            pltpu.VMEM((H_PAD,), jnp.float32),
        ],
    )(ref_a, ref_b)


def _norm_body(part_ref, out_ref):
    s = jnp.sum(part_ref[...], axis=0, keepdims=True)
    out_ref[...] = lax.rsqrt(s + 1.0)


@jax.jit
def _norm(part):
    return pl.pallas_call(
        _norm_body,
        out_shape=jax.ShapeDtypeStruct((1, H_PAD), jnp.float32),
    )(part)


def kernel(X, ref_a, ref_b):
    part = _partial_hist(ref_a, ref_b)
    normp = _norm(part)
    return normp[0, :N_NODES].reshape(N_NODES, 1)


# speedup vs baseline: 142.3188x; 1.0380x over previous
import functools

import jax
import jax.numpy as jnp
from jax import lax
from jax.experimental import pallas as pl
from jax.experimental.pallas import tpu as pltpu
from jax.experimental.pallas import tpu_sc as plsc

N_NODES = 10000
N_EDGES = 320000
NC, NS, L = 2, 16, 16
NW = NC * NS
E_PER = N_EDGES // NW
H_PAD = 10240


def _hist_body(a_hbm, b_hbm, out_hbm, a_v, b_v, hist_v, sem_a, sem_b):
    wid = lax.axis_index("s") * NC + lax.axis_index("c")
    base = wid * E_PER
    cp_a = pltpu.async_copy(a_hbm.at[pl.ds(base, E_PER)], a_v, sem_a)
    cp_b = pltpu.async_copy(b_hbm.at[pl.ds(base, E_PER)], b_v, sem_b)

    zeros = jnp.zeros((L,), jnp.float32)

    def zero_body(i, c):
        hist_v[pl.ds(i * L, L)] = zeros
        return c

    lax.fori_loop(0, H_PAD // L, zero_body, 0, unroll=8)
    cp_a.wait()
    cp_b.wait()

    ones = jnp.ones((L,), jnp.float32)
    UNROLL = 5

    def scat_body(i, c):
        off = i * (L * UNROLL)
        for u in range(UNROLL):
            plsc.addupdate_scatter(hist_v, [a_v[pl.ds(off + u * L, L)]], ones)
            plsc.addupdate_scatter(hist_v, [b_v[pl.ds(off + u * L, L)]], ones)
        return c

    lax.fori_loop(0, E_PER // (L * UNROLL), scat_body, 0)

    pltpu.sync_copy(hist_v, out_hbm.at[wid])


@jax.jit
def _partial_hist(ref_a, ref_b):
    mesh = plsc.VectorSubcoreMesh(core_axis_name="c", subcore_axis_name="s")
    return pl.kernel(
        _hist_body,
        out_type=jax.ShapeDtypeStruct((NW, H_PAD), jnp.float32),
        mesh=mesh,
        compiler_params=pltpu.CompilerParams(needs_layout_passes=False),
        scratch_types=[
            pltpu.VMEM((E_PER,), jnp.int32),
            pltpu.VMEM((E_PER,), jnp.int32),
            pltpu.VMEM((H_PAD,), jnp.float32),
            pltpu.SemaphoreType.DMA,
            pltpu.SemaphoreType.DMA,
        ],
    )(ref_a, ref_b)


def _norm_body(part_ref, out_ref):
    s = jnp.sum(part_ref[...], axis=0, keepdims=True)
    out_ref[...] = lax.rsqrt(s + 1.0)


@jax.jit
def _norm(part):
    return pl.pallas_call(
        _norm_body,
        out_shape=jax.ShapeDtypeStruct((1, H_PAD), jnp.float32),
    )(part)


def kernel(X, ref_a, ref_b):
    part = _partial_hist(ref_a, ref_b)
    normp = _norm(part)
    return normp[0, :N_NODES].reshape(N_NODES, 1)


# speedup vs baseline: 169.7010x; 1.1924x over previous
import functools

import jax
import jax.numpy as jnp
from jax import lax
from jax.experimental import pallas as pl
from jax.experimental.pallas import tpu as pltpu
from jax.experimental.pallas import tpu_sc as plsc

N_NODES = 10000
N_EDGES = 320000
NC, NS, L = 2, 16, 16
NW = NC * NS
E_PER = N_EDGES // NW
H_PAD = 10240


def _hist_body(a_hbm, b_hbm, out_hbm, a_v, b_v, hist_v, sem_a, sem_b):
    wid = lax.axis_index("s") * NC + lax.axis_index("c")
    base = wid * E_PER
    cp_a = pltpu.async_copy(a_hbm.at[pl.ds(base, E_PER)], a_v, sem_a)
    cp_b = pltpu.async_copy(b_hbm.at[pl.ds(base, E_PER)], b_v, sem_b)

    zeros = jnp.zeros((L,), jnp.float32)

    @plsc.parallel_loop(0, H_PAD // L, 1, unroll=8)
    def zero_body(i):
        hist_v[pl.ds(i * L, L)] = zeros

    cp_a.wait()
    cp_b.wait()

    ones = jnp.ones((L,), jnp.float32)

    @plsc.parallel_loop(0, E_PER // L, 1, unroll=5)
    def scat_body(i):
        off = i * L
        plsc.addupdate_scatter(hist_v, [a_v[pl.ds(off, L)]], ones)
        plsc.addupdate_scatter(hist_v, [b_v[pl.ds(off, L)]], ones)

    pltpu.sync_copy(hist_v, out_hbm.at[wid])


@jax.jit
def _partial_hist(ref_a, ref_b):
    mesh = plsc.VectorSubcoreMesh(core_axis_name="c", subcore_axis_name="s")
    return pl.kernel(
        _hist_body,
        out_type=jax.ShapeDtypeStruct((NW, H_PAD), jnp.float32),
        mesh=mesh,
        compiler_params=pltpu.CompilerParams(needs_layout_passes=False),
        scratch_types=[
            pltpu.VMEM((E_PER,), jnp.int32),
            pltpu.VMEM((E_PER,), jnp.int32),
            pltpu.VMEM((H_PAD,), jnp.float32),
            pltpu.SemaphoreType.DMA,
            pltpu.SemaphoreType.DMA,
        ],
    )(ref_a, ref_b)


def _norm_body(part_ref, out_ref):
    s = jnp.sum(part_ref[...], axis=0, keepdims=True)
    out_ref[...] = lax.rsqrt(s + 1.0)


@jax.jit
def _norm(part):
    return pl.pallas_call(
        _norm_body,
        out_shape=jax.ShapeDtypeStruct((1, H_PAD), jnp.float32),
    )(part)


def kernel(X, ref_a, ref_b):
    part = _partial_hist(ref_a, ref_b)
    normp = _norm(part)
    return normp[0, :N_NODES].reshape(N_NODES, 1)
